# Initial kernel scaffold; baseline (speedup 1.0000x reference)
import jax
import jax.numpy as jnp
from jax.experimental import pallas as pl


def kernel(queries, keys, values):
    raise NotImplementedError("write your pallas kernel here")


# speedup vs baseline: 6.1172x; 6.1172x over previous
import math

import jax
import jax.numpy as jnp
import numpy as np
from jax import lax
from jax.experimental import pallas as pl
from jax.experimental.pallas import tpu as pltpu

_L = 4096
_D = 64
_H = 12
_U = 45
_BQ = 256
_BS = 512
_NEG = float(np.float32(-3e38))

_IDX_NP = np.asarray(
    jax.random.randint(jax.random.key(42), (_L, _U), 0, _L), dtype=np.int64
)
_COUNTS_NP = np.zeros((_L, _L), np.int8)
np.add.at(_COUNTS_NP, (np.arange(_L)[:, None], _IDX_NP), 1)


def _m_body(c_ref, q_ref, k_ref, m_ref):
    h = pl.program_id(1)
    q = q_ref[0]
    k = k_ref[h]
    s = lax.dot_general(q, k, (((1,), (1,)), ((), ())),
                        preferred_element_type=jnp.float32, precision=lax.Precision.HIGHEST)
    c = c_ref[...].astype(jnp.float32)
    mx = jnp.max(jnp.where(c > 0.5, s, _NEG), axis=1, keepdims=True)
    sm = jnp.sum(s * c, axis=1, keepdims=True)
    m_ref[0, 0] = mx - sm * (1.0 / _L)


def _topk_body(m_ref, idx_ref):
    m = m_ref[...]
    iota = lax.broadcasted_iota(jnp.int32, (_H, _L), 1)
    lane = lax.broadcasted_iota(jnp.int32, (_H, 64), 1)
    big = jnp.int32(1 << 30)

    def body(i, carry):
        mm, acc = carry
        rmax = jnp.max(mm, axis=1, keepdims=True)
        cand = jnp.where(mm == rmax, iota, big)
        amin = jnp.min(cand, axis=1, keepdims=True)
        mm = jnp.where(iota == amin, _NEG, mm)
        acc = jnp.where(lane == i, amin, acc)
        return mm, acc

    _, acc = lax.fori_loop(0, _U, body, (m, jnp.full((_H, 64), big, jnp.int32)))
    idx_ref[:, 0, :] = acc


def _ctx_body(q_ref, k_ref, v_ref, idx_ref, ctx_ref, upd_s, carry_s):
    h = pl.program_id(0)
    sb = pl.program_id(1)
    idxh = idx_ref[h]

    @pl.when(sb == 0)
    def _init():
        io_l = lax.broadcasted_iota(jnp.int32, (_L, 64), 0)
        one_t = (io_l == idxh).astype(jnp.float32)
        qr = lax.dot_general(one_t, q_ref[0], (((0,), (0,)), ((), ())),
                             preferred_element_type=jnp.float32, precision=lax.Precision.HIGHEST)
        s = lax.dot_general(qr, k_ref[0], (((1,), (1,)), ((), ())),
                            preferred_element_type=jnp.float32, precision=lax.Precision.HIGHEST)
        s = s * (1.0 / math.sqrt(_D))
        smax = jnp.max(s, axis=1, keepdims=True)
        e = jnp.exp(s - smax)
        p = e / jnp.sum(e, axis=1, keepdims=True)
        upd_s[...] = lax.dot_general(p, v_ref[0], (((1,), (0,)), ((), ())),
                                     preferred_element_type=jnp.float32, precision=lax.Precision.HIGHEST)
        carry_s[...] = jnp.zeros((8, _D), jnp.float32)

    vblk = v_ref[0, pl.ds(sb * _BS, _BS), :]
    io_r = lax.broadcasted_iota(jnp.int32, (_BS, _BS), 0)
    io_c = lax.broadcasted_iota(jnp.int32, (_BS, _BS), 1)
    tri = (io_r >= io_c).astype(jnp.float32)
    csum = lax.dot_general(tri, vblk, (((1,), (0,)), ((), ())),
                           preferred_element_type=jnp.float32, precision=lax.Precision.HIGHEST)
    csum = csum + carry_s[0:1, :]
    carry_s[0:1, :] = carry_s[0:1, :] + jnp.sum(vblk, axis=0, keepdims=True)

    gio = lax.broadcasted_iota(jnp.int32, (_BS, 1), 0) + sb * _BS
    den = (gio + 1).astype(jnp.float32)
    cmean = csum / den

    oh = (gio == idxh).astype(jnp.float32)
    repl = lax.dot_general(oh, upd_s[...], (((1,), (0,)), ((), ())),
                           preferred_element_type=jnp.float32, precision=lax.Precision.HIGHEST)
    hit = jnp.sum(oh, axis=1, keepdims=True) > 0.5
    ctx_ref[0] = jnp.where(hit, repl, cmean)


def kernel(queries, keys, values):
    B, H, L, D = queries.shape
    assert (B, H, L, D) == (1, _H, _L, _D)
    q = queries[0]
    k = keys[0]
    v = values[0]

    nq = _L // _BQ
    m4 = pl.pallas_call(
        _m_body,
        grid=(nq, _H),
        in_specs=[
            pl.BlockSpec((_BQ, _L), lambda qb, h: (qb, 0)),
            pl.BlockSpec((1, _BQ, _D), lambda qb, h: (h, qb, 0)),
            pl.BlockSpec((_H, _L, _D), lambda qb, h: (0, 0, 0)),
        ],
        out_specs=pl.BlockSpec((1, 1, _BQ, 1), lambda qb, h: (h, qb, 0, 0)),
        out_shape=jax.ShapeDtypeStruct((_H, nq, _BQ, 1), jnp.float32),
    )(_COUNTS_NP, q, k)
    m2 = m4.reshape(_H, _L)

    idx = pl.pallas_call(
        _topk_body,
        out_shape=jax.ShapeDtypeStruct((_H, 1, 64), jnp.int32),
    )(m2)

    ns = _L // _BS
    ctx = pl.pallas_call(
        _ctx_body,
        grid=(_H, ns),
        in_specs=[
            pl.BlockSpec((1, _L, _D), lambda h, sb: (h, 0, 0)),
            pl.BlockSpec((1, _L, _D), lambda h, sb: (h, 0, 0)),
            pl.BlockSpec((1, _L, _D), lambda h, sb: (h, 0, 0)),
            pl.BlockSpec((_H, 1, 64), lambda h, sb: (0, 0, 0)),
        ],
        out_specs=pl.BlockSpec((1, _BS, _D), lambda h, sb: (h, sb, 0)),
        out_shape=jax.ShapeDtypeStruct((_H, _L, _D), jnp.float32),
        scratch_shapes=[
            pltpu.VMEM((64, _D), jnp.float32),
            pltpu.VMEM((8, _D), jnp.float32),
        ],
    )(q, k, v, idx)
    return ctx[None]


# speedup vs baseline: 6.7432x; 1.1023x over previous
import math

import jax
import jax.numpy as jnp
import numpy as np
from jax import lax
from jax.experimental import pallas as pl
from jax.experimental.pallas import tpu as pltpu

_L = 4096
_D = 64
_H = 12
_U = 45
_BQ = 512
_BS = 512
_NEG = float(np.float32(-3e38))

try:
    _IDX_NP = np.asarray(
        jax.random.randint(jax.random.key(42), (_L, _U), 0, _L), dtype=np.int64
    )
    _COUNTS_T_NP = np.zeros((_L, _L), np.float32)
    np.add.at(_COUNTS_T_NP, (_IDX_NP, np.arange(_L)[:, None]), 1.0)
except Exception:
    _COUNTS_T_NP = None

_TRI_NP = np.tril(np.ones((128, 128), np.float32))


def _sample_counts_t():
    if _COUNTS_T_NP is not None:
        return _COUNTS_T_NP
    idx = jax.random.randint(jax.random.key(42), (_L, _U), 0, _L)
    z = jnp.zeros((_L, _L), jnp.float32)
    return z.at[idx, jnp.arange(_L)[:, None]].add(1.0)


def _bf16_split(x):
    hi = x.astype(jnp.bfloat16)
    lo = (x - hi.astype(jnp.float32)).astype(jnp.bfloat16)
    return hi, lo


def _dot3(a, b, dims):
    ah, al = _bf16_split(a)
    bh, bl = _bf16_split(b)
    dn = (dims, ((), ()))
    out = lax.dot_general(ah, bh, dn, preferred_element_type=jnp.float32)
    out += lax.dot_general(ah, bl, dn, preferred_element_type=jnp.float32)
    out += lax.dot_general(al, bh, dn, preferred_element_type=jnp.float32)
    return out


def _dot_exact_lhs(a_bf, b, dims):
    bh, bl = _bf16_split(b)
    dn = (dims, ((), ()))
    out = lax.dot_general(a_bf, bh, dn, preferred_element_type=jnp.float32)
    out += lax.dot_general(a_bf, bl, dn, preferred_element_type=jnp.float32)
    return out


def _m_body(c_ref, q_ref, k_ref, m_ref):
    h = pl.program_id(1)
    q = q_ref[0, 0]
    k = k_ref[0, h]
    st = lax.dot_general(k, q, (((1,), (1,)), ((), ())),
                         preferred_element_type=jnp.float32,
                         precision=lax.Precision.HIGHEST)
    c = c_ref[...]
    mx = jnp.max(jnp.where(c > 0.5, st, _NEG), axis=0, keepdims=True)
    sm = jnp.sum(st * c, axis=0, keepdims=True)
    m_ref[0] = mx - sm * (1.0 / _L)


def _topk_body(m_ref, idx_ref):
    m = m_ref[:, 0, :]
    iota = lax.broadcasted_iota(jnp.int32, (_H, _L), 1)
    lane = lax.broadcasted_iota(jnp.int32, (_H, 64), 1)
    big = jnp.int32(1 << 30)

    def body(i, carry):
        mm, acc = carry
        rmax = jnp.max(mm, axis=1, keepdims=True)
        cand = jnp.where(mm == rmax, iota, big)
        amin = jnp.min(cand, axis=1, keepdims=True)
        mm = jnp.where(iota == amin, _NEG, mm)
        acc = jnp.where(lane == i, amin, acc)
        return mm, acc

    _, acc = lax.fori_loop(0, _U, body, (m, jnp.full((_H, 64), big, jnp.int32)))
    idx_ref[:, 0, :] = acc


def _ctx_body(q_ref, k_ref, v_ref, idx_ref, tri_ref, ctx_ref, upd_s, carry_s):
    h = pl.program_id(0)
    sb = pl.program_id(1)
    idxh = idx_ref[h]

    @pl.when(sb == 0)
    def _init():
        io_l = lax.broadcasted_iota(jnp.int32, (_L, 64), 0)
        one_t = (io_l == idxh).astype(jnp.bfloat16)
        qr = _dot_exact_lhs(one_t, q_ref[0, 0], ((0,), (0,)))
        s = _dot3(qr, k_ref[0, 0], ((1,), (1,)))
        s = s * (1.0 / math.sqrt(_D))
        smax = jnp.max(s, axis=1, keepdims=True)
        e = jnp.exp(s - smax)
        p = e / jnp.sum(e, axis=1, keepdims=True)
        upd_s[...] = _dot3(p, v_ref[0, 0], ((1,), (0,)))
        carry_s[...] = jnp.zeros((8, _D), jnp.float32)

    vblk = v_ref[0, 0, pl.ds(sb * _BS, _BS), :]
    tri = tri_ref[...].astype(jnp.bfloat16)
    run = carry_s[0:1, :]
    outs = []
    for j in range(_BS // 128):
        vj = vblk[j * 128:(j + 1) * 128]
        cj = _dot_exact_lhs(tri, vj, ((1,), (0,)))
        outs.append(cj + run)
        run = run + jnp.sum(vj, axis=0, keepdims=True)
    carry_s[0:1, :] = run
    csum = jnp.concatenate(outs, axis=0)

    gio = lax.broadcasted_iota(jnp.int32, (_BS, 1), 0) + sb * _BS
    den = (gio + 1).astype(jnp.float32)
    cmean = csum / den

    oh = (gio == idxh).astype(jnp.bfloat16)
    repl = _dot_exact_lhs(oh, upd_s[...], ((1,), (0,)))
    hit = jnp.sum(oh.astype(jnp.float32), axis=1, keepdims=True) > 0.5
    ctx_ref[0, 0] = jnp.where(hit, repl, cmean)


def kernel(queries, keys, values):
    B, H, L, D = queries.shape
    assert (B, H, L, D) == (1, _H, _L, _D)

    nq = _L // _BQ
    m3 = pl.pallas_call(
        _m_body,
        grid=(nq, _H),
        in_specs=[
            pl.BlockSpec((_L, _BQ), lambda qb, h: (0, qb)),
            pl.BlockSpec((1, 1, _BQ, _D), lambda qb, h: (0, h, qb, 0)),
            pl.BlockSpec((1, _H, _L, _D), lambda qb, h: (0, 0, 0, 0)),
        ],
        out_specs=pl.BlockSpec((1, 1, _BQ), lambda qb, h: (h, 0, qb)),
        out_shape=jax.ShapeDtypeStruct((_H, 1, _L), jnp.float32),
    )(_sample_counts_t(), queries, keys)

    idx = pl.pallas_call(
        _topk_body,
        out_shape=jax.ShapeDtypeStruct((_H, 1, 64), jnp.int32),
    )(m3)

    ns = _L // _BS
    ctx = pl.pallas_call(
        _ctx_body,
        grid=(_H, ns),
        in_specs=[
            pl.BlockSpec((1, 1, _L, _D), lambda h, sb: (0, h, 0, 0)),
            pl.BlockSpec((1, 1, _L, _D), lambda h, sb: (0, h, 0, 0)),
            pl.BlockSpec((1, 1, _L, _D), lambda h, sb: (0, h, 0, 0)),
            pl.BlockSpec((_H, 1, 64), lambda h, sb: (0, 0, 0)),
            pl.BlockSpec((128, 128), lambda h, sb: (0, 0)),
        ],
        out_specs=pl.BlockSpec((1, 1, _BS, _D), lambda h, sb: (0, h, sb, 0)),
        out_shape=jax.ShapeDtypeStruct((1, _H, _L, _D), jnp.float32),
        scratch_shapes=[
            pltpu.VMEM((64, _D), jnp.float32),
            pltpu.VMEM((8, _D), jnp.float32),
        ],
    )(queries, keys, values, idx, _TRI_NP)
    return ctx


# speedup vs baseline: 12.0831x; 1.7919x over previous
import math

import jax
import jax.numpy as jnp
import numpy as np
from jax import lax
from jax.experimental import pallas as pl
from jax.experimental.pallas import tpu as pltpu

_L = 4096
_D = 64
_H = 12
_U = 45
_BQ = 512
_BS = 512
_NEG = float(np.float32(-3e38))

try:
    _IDX_NP = np.asarray(
        jax.random.randint(jax.random.key(42), (_L, _U), 0, _L), dtype=np.int64
    )
    _COUNTS_T_NP = np.zeros((_L, _L), np.float32)
    np.add.at(_COUNTS_T_NP, (_IDX_NP, np.arange(_L)[:, None]), 1.0)
except Exception:
    _COUNTS_T_NP = None

_TRI_NP = np.tril(np.ones((128, 128), np.float32))


def _sample_counts_t():
    if _COUNTS_T_NP is not None:
        return _COUNTS_T_NP
    idx = jax.random.randint(jax.random.key(42), (_L, _U), 0, _L)
    z = jnp.zeros((_L, _L), jnp.float32)
    return z.at[idx, jnp.arange(_L)[:, None]].add(1.0)


def _bf16_split(x):
    hi = x.astype(jnp.bfloat16)
    lo = (x - hi.astype(jnp.float32)).astype(jnp.bfloat16)
    return hi, lo


def _dot3(a, b, dims):
    ah, al = _bf16_split(a)
    bh, bl = _bf16_split(b)
    dn = (dims, ((), ()))
    out = lax.dot_general(ah, bh, dn, preferred_element_type=jnp.float32)
    out += lax.dot_general(ah, bl, dn, preferred_element_type=jnp.float32)
    out += lax.dot_general(al, bh, dn, preferred_element_type=jnp.float32)
    return out


def _dot_exact_lhs(a_bf, b, dims):
    bh, bl = _bf16_split(b)
    dn = (dims, ((), ()))
    out = lax.dot_general(a_bf, bh, dn, preferred_element_type=jnp.float32)
    out += lax.dot_general(a_bf, bl, dn, preferred_element_type=jnp.float32)
    return out


def _split3(x):
    h1 = x.astype(jnp.bfloat16)
    r1 = x - h1.astype(jnp.float32)
    h2 = r1.astype(jnp.bfloat16)
    h3 = (r1 - h2.astype(jnp.float32)).astype(jnp.bfloat16)
    return h1, h2, h3


def _m_body(c_ref, q_ref, k_ref, m_ref):
    h = pl.program_id(1)
    q = q_ref[0, 0]
    k = k_ref[0, h]
    qh, qm, ql = _split3(q)
    kh, km, kl = _split3(k)
    dn = (((1,), (1,)), ((), ()))
    st = lax.dot_general(jnp.concatenate([kh, kh, km], axis=1),
                         jnp.concatenate([qh, qm, qh], axis=1),
                         dn, preferred_element_type=jnp.float32)
    st = st + lax.dot_general(jnp.concatenate([kh, km, kl], axis=1),
                              jnp.concatenate([ql, qm, qh], axis=1),
                              dn, preferred_element_type=jnp.float32)
    c = c_ref[...]
    mx = jnp.max(jnp.where(c > 0.5, st, _NEG), axis=0, keepdims=True)
    sm = jnp.sum(st * c, axis=0, keepdims=True)
    m_ref[0] = mx - sm * (1.0 / _L)


def _topk_body(m_ref, idx_ref):
    m = m_ref[:, 0, :]
    iota = lax.broadcasted_iota(jnp.int32, (_H, _L), 1)
    lane = lax.broadcasted_iota(jnp.int32, (_H, 64), 1)
    big = jnp.int32(1 << 30)

    def body(i, carry):
        mm, acc = carry
        rmax = jnp.max(mm, axis=1, keepdims=True)
        cand = jnp.where(mm == rmax, iota, big)
        amin = jnp.min(cand, axis=1, keepdims=True)
        mm = jnp.where(iota == amin, _NEG, mm)
        acc = jnp.where(lane == i, amin, acc)
        return mm, acc

    _, acc = lax.fori_loop(0, _U, body, (m, jnp.full((_H, 64), big, jnp.int32)))
    idx_ref[:, 0, :] = acc


def _ctx_body(q_ref, k_ref, v_ref, idx_ref, tri_ref, ctx_ref, upd_s, carry_s):
    h = pl.program_id(0)
    sb = pl.program_id(1)
    idxh = idx_ref[h]

    @pl.when(sb == 0)
    def _init():
        io_l = lax.broadcasted_iota(jnp.int32, (_L, 64), 0)
        one_t = (io_l == idxh).astype(jnp.bfloat16)
        qr = _dot_exact_lhs(one_t, q_ref[0, 0], ((0,), (0,)))
        s = _dot3(qr, k_ref[0, 0], ((1,), (1,)))
        s = s * (1.0 / math.sqrt(_D))
        smax = jnp.max(s, axis=1, keepdims=True)
        e = jnp.exp(s - smax)
        p = e / jnp.sum(e, axis=1, keepdims=True)
        upd_s[...] = _dot3(p, v_ref[0, 0], ((1,), (0,)))
        carry_s[...] = jnp.zeros((8, _D), jnp.float32)

    vblk = v_ref[0, 0, pl.ds(sb * _BS, _BS), :]
    tri = tri_ref[...].astype(jnp.bfloat16)
    run = carry_s[0:1, :]
    outs = []
    for j in range(_BS // 128):
        vj = vblk[j * 128:(j + 1) * 128]
        cj = _dot_exact_lhs(tri, vj, ((1,), (0,)))
        outs.append(cj + run)
        run = run + jnp.sum(vj, axis=0, keepdims=True)
    carry_s[0:1, :] = run
    csum = jnp.concatenate(outs, axis=0)

    gio = lax.broadcasted_iota(jnp.int32, (_BS, 1), 0) + sb * _BS
    den = (gio + 1).astype(jnp.float32)
    cmean = csum / den

    oh = (gio == idxh).astype(jnp.bfloat16)
    repl = _dot_exact_lhs(oh, upd_s[...], ((1,), (0,)))
    hit = jnp.sum(oh.astype(jnp.float32), axis=1, keepdims=True) > 0.5
    ctx_ref[0, 0] = jnp.where(hit, repl, cmean)


def kernel(queries, keys, values):
    B, H, L, D = queries.shape
    assert (B, H, L, D) == (1, _H, _L, _D)

    nq = _L // _BQ
    m3 = pl.pallas_call(
        _m_body,
        grid=(nq, _H),
        in_specs=[
            pl.BlockSpec((_L, _BQ), lambda qb, h: (0, qb)),
            pl.BlockSpec((1, 1, _BQ, _D), lambda qb, h: (0, h, qb, 0)),
            pl.BlockSpec((1, _H, _L, _D), lambda qb, h: (0, 0, 0, 0)),
        ],
        out_specs=pl.BlockSpec((1, 1, _BQ), lambda qb, h: (h, 0, qb)),
        out_shape=jax.ShapeDtypeStruct((_H, 1, _L), jnp.float32),
    )(_sample_counts_t(), queries, keys)

    idx = pl.pallas_call(
        _topk_body,
        out_shape=jax.ShapeDtypeStruct((_H, 1, 64), jnp.int32),
    )(m3)

    ns = _L // _BS
    ctx = pl.pallas_call(
        _ctx_body,
        grid=(_H, ns),
        in_specs=[
            pl.BlockSpec((1, 1, _L, _D), lambda h, sb: (0, h, 0, 0)),
            pl.BlockSpec((1, 1, _L, _D), lambda h, sb: (0, h, 0, 0)),
            pl.BlockSpec((1, 1, _L, _D), lambda h, sb: (0, h, 0, 0)),
            pl.BlockSpec((_H, 1, 64), lambda h, sb: (0, 0, 0)),
            pl.BlockSpec((128, 128), lambda h, sb: (0, 0)),
        ],
        out_specs=pl.BlockSpec((1, 1, _BS, _D), lambda h, sb: (0, h, sb, 0)),
        out_shape=jax.ShapeDtypeStruct((1, _H, _L, _D), jnp.float32),
        scratch_shapes=[
            pltpu.VMEM((64, _D), jnp.float32),
            pltpu.VMEM((8, _D), jnp.float32),
        ],
    )(queries, keys, values, idx, _TRI_NP)
    return ctx
